# baseline (speedup 1.0000x reference)
import jax
import jax.numpy as jnp
from jax import lax
from jax.experimental import pallas as pl
from jax.experimental.pallas import tpu as pltpu

_TB = 256


def _acc_kernel(lens_ref, x_ref, tgt_ref, out_ref, c_ref, m_ref):
    b = pl.program_id(0)
    j = pl.program_id(1)
    nb = pl.num_programs(0)
    nj = pl.num_programs(1)
    x = x_ref[...]
    TB, V = x.shape
    rowmax = jnp.max(x, axis=-1, keepdims=True)
    idx = lax.broadcasted_iota(jnp.int32, x.shape, 1)
    pred = jnp.min(jnp.where(x == rowmax, idx, V), axis=-1)
    tgt = tgt_ref[0, 0]
    blen = lens_ref[b] + 1
    t_idx = lax.broadcasted_iota(jnp.int32, (TB,), 0) + j * TB
    mask = t_idx < blen
    c_part = jnp.sum(jnp.where(mask & (pred == tgt), 1.0, 0.0))
    m_part = jnp.sum(jnp.where(mask, 1.0, 0.0))

    @pl.when((b == 0) & (j == 0))
    def _init():
        c_ref[0] = 0.0
        m_ref[0] = 0.0

    @pl.when(j * TB < blen)
    def _acc():
        c_ref[0] += c_part
        m_ref[0] += m_part

    @pl.when((b == nb - 1) & (j == nj - 1))
    def _fin():
        out_ref[0] = c_ref[0] / m_ref[0]


def kernel(outputs, tokens, tokens_lens):
    T, B, V = outputs.shape
    nj = T // _TB
    x2d = outputs.reshape(T, B * V)
    tgt = jnp.concatenate([tokens[1:], tokens[-1:]], axis=0)
    tgt3 = tgt.T.reshape(B * nj, 1, _TB)

    def x_map(b, j, lens):
        return (j, b)

    def tgt_map(b, j, lens):
        return (b * nj + j, 0, 0)

    grid_spec = pltpu.PrefetchScalarGridSpec(
        num_scalar_prefetch=1,
        grid=(B, nj),
        in_specs=[
            pl.BlockSpec((_TB, V), x_map),
            pl.BlockSpec((1, 1, _TB), tgt_map),
        ],
        out_specs=pl.BlockSpec(memory_space=pltpu.SMEM),
        scratch_shapes=[
            pltpu.SMEM((1,), jnp.float32),
            pltpu.SMEM((1,), jnp.float32),
        ],
    )
    acc = pl.pallas_call(
        _acc_kernel,
        grid_spec=grid_spec,
        out_shape=jax.ShapeDtypeStruct((1,), jnp.float32),
        compiler_params=pltpu.CompilerParams(
            dimension_semantics=("arbitrary", "arbitrary"),
        ),
    )(tokens_lens, x2d, tgt3)
    return acc[0]


# speedup vs baseline: 1.9766x; 1.9766x over previous
import functools

import jax
import jax.numpy as jnp
from jax import lax
from jax.experimental import pallas as pl
from jax.experimental.pallas import tpu as pltpu
from jax.experimental.pallas import tpu_sc as plsc

_TB = 256
_NW = 32
_WPC = 4


def _tc_kernel(aux_ref, x_ref, tgt_ref, out_ref, c_ref, m_ref):
    j = pl.program_id(0)
    nj = pl.num_programs(0)
    x = x_ref[...]
    TB, B, V = x.shape
    rowmax = jnp.max(x, axis=-1, keepdims=True)
    idx = lax.broadcasted_iota(jnp.int32, x.shape, 2)
    pred = jnp.min(jnp.where(x == rowmax, idx, V), axis=-1)
    tgt = tgt_ref[0]
    s = aux_ref[8]
    t_idx = lax.broadcasted_iota(jnp.int32, (TB, B), 0) + j * TB
    b_idx = lax.broadcasted_iota(jnp.int32, (TB, B), 1)
    lens_v = jnp.zeros((TB, B), jnp.int32)
    for b in range(B):
        lens_v = jnp.where(b_idx == b, aux_ref[b], lens_v)
    mask = t_idx < jnp.minimum(lens_v, s)
    c_part = jnp.sum(jnp.where(mask & (pred == tgt), 1.0, 0.0))
    m_part = jnp.sum(jnp.where(mask, 1.0, 0.0))

    @pl.when(j == 0)
    def _init():
        c_ref[0] = 0.0
        m_ref[0] = 0.0

    @pl.when(j * TB < s)
    def _acc():
        c_ref[0] += c_part
        m_ref[0] += m_part

    @pl.when(j == nj - 1)
    def _fin():
        out_ref[0] = c_ref[0]
        out_ref[1] = m_ref[0]


def _tc_counts(outputs, tgt, aux_tc):
    T, B, V = outputs.shape
    nj = T // _TB
    tgt3 = tgt.reshape(nj, _TB, B)

    def x_map(j, aux):
        return (jnp.minimum(j, lax.div(aux[8], _TB) - 1), 0, 0)

    def tgt_map(j, aux):
        return (jnp.minimum(j, lax.div(aux[8], _TB) - 1), 0, 0)

    grid_spec = pltpu.PrefetchScalarGridSpec(
        num_scalar_prefetch=1,
        grid=(nj,),
        in_specs=[
            pl.BlockSpec((_TB, B, V), x_map),
            pl.BlockSpec((1, _TB, B), tgt_map),
        ],
        out_specs=pl.BlockSpec(memory_space=pltpu.SMEM),
        scratch_shapes=[
            pltpu.SMEM((1,), jnp.float32),
            pltpu.SMEM((1,), jnp.float32),
        ],
    )
    return pl.pallas_call(
        _tc_kernel,
        grid_spec=grid_spec,
        out_shape=jax.ShapeDtypeStruct((2,), jnp.float32),
        compiler_params=pltpu.CompilerParams(
            dimension_semantics=("arbitrary",),
        ),
    )(aux_tc, outputs, tgt3)


def _all_lanes(x, op):
    io = lax.broadcasted_iota(jnp.int32, (16,), 0)
    dnums = lax.GatherDimensionNumbers(
        offset_dims=(), collapsed_slice_dims=(0,), start_index_map=(0,)
    )
    for sh in (8, 4, 2, 1):
        idx = ((io + sh) & 15)[:, None]
        rot = lax.gather(
            x, idx, dnums, (1,),
            mode=lax.GatherScatterMode.PROMISE_IN_BOUNDS,
        )
        x = op(x, rot)
    return x


def _sc_counts(x2d, tgt_pad, aux, T, B, V):
    nchunk = V // 16

    mesh = plsc.VectorSubcoreMesh(core_axis_name="c", subcore_axis_name="s")

    @functools.partial(
        pl.kernel,
        mesh=mesh,
        out_type=[
            jax.ShapeDtypeStruct((_NW, 16), jnp.int32),
            jax.ShapeDtypeStruct((_NW, 16), jnp.int32),
        ],
        scratch_types=[
            pltpu.VMEM((16, V), jnp.float32),
            pltpu.VMEM((T + 16,), jnp.int32),
            pltpu.VMEM((32,), jnp.int32),
            pltpu.VMEM((16,), jnp.int32),
            pltpu.VMEM((16,), jnp.int32),
            pltpu.SemaphoreType.DMA,
        ],
    )
    def sck(x_hbm, tgtp_hbm, aux_hbm, cc_out, vc_out,
            rows_v, tgtbuf, auxbuf, ccbuf, vcbuf, sem):
        wid = lax.axis_index("s") * 2 + lax.axis_index("c")
        b = wid // _WPC
        q = wid % _WPC
        io = lax.broadcasted_iota(jnp.int32, (16,), 0)

        pltpu.sync_copy(aux_hbm, auxbuf)
        len_b = auxbuf[pl.ds(b, 16)][0]
        s_b = auxbuf[pl.ds(8 + b, 16)][0]

        span = jnp.maximum(len_b - s_b, 0)
        per_q = lax.div(span + (_WPC - 1), _WPC)
        t0 = s_b + q * per_q
        t1 = jnp.minimum(t0 + per_q, len_b)
        ngroups = lax.div(jnp.maximum(t1 - t0, 0) + 15, 16)

        pltpu.sync_copy(tgtp_hbm.at[b], tgtbuf)

        def gbody(gi, carry):
            cc, vc = carry
            t = t0 + gi * 16
            tvec = t + io
            tcl = jnp.minimum(tvec, t1 - 1)
            ridx = tcl * B + b
            pltpu.async_copy(x_hbm.at[ridx], rows_v, sem).wait()
            preds = jnp.zeros((16,), jnp.int32)
            for g in range(16):
                def chunk_body(jj, carry2):
                    m, bi = carry2
                    for u in range(4):
                        j = jj * 4 + u
                        v = rows_v[g, pl.ds(j * 16, 16)]
                        upd = v > m
                        bi = jnp.where(upd, j, bi)
                        m = jnp.maximum(m, v)
                    return m, bi
                m0 = jnp.full((16,), -jnp.inf, jnp.float32)
                bi0 = jnp.zeros((16,), jnp.int32)
                m, bi = lax.fori_loop(0, nchunk // 4, chunk_body, (m0, bi0))
                rm = _all_lanes(m, jnp.maximum)
                cand = jnp.where(
                    m == rm, (bi * 16 + io).astype(jnp.float32), float(V)
                )
                p = _all_lanes(cand, jnp.minimum).astype(jnp.int32)
                preds = jnp.where(io == g, p, preds)
            tg = tgtbuf[pl.ds(t, 16)]
            valid = tvec < t1
            cc = cc + jnp.where(valid & (preds == tg), 1, 0)
            vc = vc + jnp.where(valid, 1, 0)
            return cc, vc

        z = jnp.zeros((16,), jnp.int32)
        cc, vc = lax.fori_loop(0, ngroups, gbody, (z, z))
        ccbuf[...] = cc
        vcbuf[...] = vc
        pltpu.sync_copy(ccbuf, cc_out.at[wid])
        pltpu.sync_copy(vcbuf, vc_out.at[wid])

    return sck(x2d, tgt_pad, aux)


_R_TC = 2.8e12
_R_SC = 1.2e12
_SC_FIXED = 5e-6


def kernel(outputs, tokens, tokens_lens):
    T, B, V = outputs.shape
    lens = (tokens_lens + 1).astype(jnp.int32)
    tgt = jnp.concatenate([tokens[1:], tokens[-1:]], axis=0)

    cands = jnp.arange(_TB, T + 1, _TB, dtype=jnp.int32)
    tc_cost = cands.astype(jnp.float32) * (B * V * 4) / _R_TC
    sc_rows = jnp.sum(
        jnp.maximum(lens[None, :] - cands[:, None], 0), axis=1
    ).astype(jnp.float32)
    sc_cost = jnp.where(sc_rows > 0, sc_rows * (V * 4) / _R_SC + _SC_FIXED, 0.0)
    s = cands[jnp.argmin(jnp.maximum(tc_cost, sc_cost))]

    aux_tc = jnp.concatenate(
        [lens, s[None], jnp.zeros((7,), jnp.int32)]
    )
    aux_sc = jnp.concatenate(
        [lens, jnp.minimum(s, lens), jnp.zeros((16,), jnp.int32)]
    )

    x2d = outputs.reshape(T * B, V)
    tgt_pad = jnp.concatenate(
        [tgt.T, jnp.zeros((B, 16), jnp.int32)], axis=1
    )

    tc = _tc_counts(outputs, tgt, aux_tc)
    cc, vc = _sc_counts(x2d, tgt_pad, aux_sc, T, B, V)

    c = tc[0] + jnp.sum(cc).astype(jnp.float32)
    m = tc[1] + jnp.sum(vc).astype(jnp.float32)
    return c / m


# speedup vs baseline: 2.2204x; 1.1233x over previous
import functools

import jax
import jax.numpy as jnp
from jax import lax
from jax.experimental import pallas as pl
from jax.experimental.pallas import tpu as pltpu
from jax.experimental.pallas import tpu_sc as plsc

_TB = 256
_NW = 32
_WPC = 4


def _tc_kernel(aux_ref, x_ref, tgt_ref, out_ref, c_ref, m_ref):
    j = pl.program_id(0)
    nj = pl.num_programs(0)
    x = x_ref[...]
    TB, B, V = x.shape
    rowmax = jnp.max(x, axis=-1, keepdims=True)
    idx = lax.broadcasted_iota(jnp.int32, x.shape, 2)
    pred = jnp.min(jnp.where(x == rowmax, idx, V), axis=-1)
    tgt = tgt_ref[0]
    s = aux_ref[8]
    t_idx = lax.broadcasted_iota(jnp.int32, (TB, B), 0) + j * TB
    b_idx = lax.broadcasted_iota(jnp.int32, (TB, B), 1)
    lens_v = jnp.zeros((TB, B), jnp.int32)
    for b in range(B):
        lens_v = jnp.where(b_idx == b, aux_ref[b], lens_v)
    mask = t_idx < jnp.minimum(lens_v, s)
    c_part = jnp.sum(jnp.where(mask & (pred == tgt), 1.0, 0.0))
    m_part = jnp.sum(jnp.where(mask, 1.0, 0.0))

    @pl.when(j == 0)
    def _init():
        c_ref[0] = 0.0
        m_ref[0] = 0.0

    @pl.when(j * TB < s)
    def _acc():
        c_ref[0] += c_part
        m_ref[0] += m_part

    @pl.when(j == nj - 1)
    def _fin():
        out_ref[0] = c_ref[0]
        out_ref[1] = m_ref[0]


def _tc_counts(outputs, tgt, aux_tc):
    T, B, V = outputs.shape
    nj = T // _TB
    tgt3 = tgt.reshape(nj, _TB, B)

    def x_map(j, aux):
        return (jnp.minimum(j, lax.div(aux[8], _TB) - 1), 0, 0)

    def tgt_map(j, aux):
        return (jnp.minimum(j, lax.div(aux[8], _TB) - 1), 0, 0)

    grid_spec = pltpu.PrefetchScalarGridSpec(
        num_scalar_prefetch=1,
        grid=(nj,),
        in_specs=[
            pl.BlockSpec((_TB, B, V), x_map),
            pl.BlockSpec((1, _TB, B), tgt_map),
        ],
        out_specs=pl.BlockSpec(memory_space=pltpu.SMEM),
        scratch_shapes=[
            pltpu.SMEM((1,), jnp.float32),
            pltpu.SMEM((1,), jnp.float32),
        ],
    )
    return pl.pallas_call(
        _tc_kernel,
        grid_spec=grid_spec,
        out_shape=jax.ShapeDtypeStruct((2,), jnp.float32),
        compiler_params=pltpu.CompilerParams(
            dimension_semantics=("arbitrary",),
        ),
    )(aux_tc, outputs, tgt3)


def _all_lanes(x, op):
    io = lax.broadcasted_iota(jnp.int32, (16,), 0)
    dnums = lax.GatherDimensionNumbers(
        offset_dims=(), collapsed_slice_dims=(0,), start_index_map=(0,)
    )
    for sh in (8, 4, 2, 1):
        idx = ((io + sh) & 15)[:, None]
        rot = lax.gather(
            x, idx, dnums, (1,),
            mode=lax.GatherScatterMode.PROMISE_IN_BOUNDS,
        )
        x = op(x, rot)
    return x


def _sc_counts(x2d, tgt_pad, aux, T, B, V):
    nchunk = V // 16

    mesh = plsc.VectorSubcoreMesh(core_axis_name="c", subcore_axis_name="s")

    @functools.partial(
        pl.kernel,
        mesh=mesh,
        out_type=[
            jax.ShapeDtypeStruct((_NW, 16), jnp.int32),
            jax.ShapeDtypeStruct((_NW, 16), jnp.int32),
        ],
        scratch_types=[
            pltpu.VMEM((16, V), jnp.float32),
            pltpu.VMEM((16, V), jnp.float32),
            pltpu.VMEM((T + 16,), jnp.int32),
            pltpu.VMEM((32,), jnp.int32),
            pltpu.VMEM((16,), jnp.int32),
            pltpu.VMEM((16,), jnp.int32),
            pltpu.SemaphoreType.DMA,
            pltpu.SemaphoreType.DMA,
        ],
    )
    def sck(x_hbm, tgtp_hbm, aux_hbm, cc_out, vc_out,
            rows_a, rows_b, tgtbuf, auxbuf, ccbuf, vcbuf, sem_a, sem_b):
        wid = lax.axis_index("s") * 2 + lax.axis_index("c")
        b = wid // _WPC
        q = wid % _WPC
        io = lax.broadcasted_iota(jnp.int32, (16,), 0)

        pltpu.sync_copy(aux_hbm, auxbuf)
        len_b = auxbuf[pl.ds(b, 16)][0]
        s_b = auxbuf[pl.ds(8 + b, 16)][0]

        span = jnp.maximum(len_b - s_b, 0)
        per_q = lax.div(span + (_WPC - 1), _WPC)
        t0 = s_b + q * per_q
        t1 = jnp.minimum(t0 + per_q, len_b)
        ngroups = lax.div(jnp.maximum(t1 - t0, 0) + 15, 16)

        def ridx_of(g):
            tv = t0 + g * 16 + io
            return jnp.minimum(tv, t1 - 1) * B + b

        def start(g, buf, sem):
            gcl = jnp.minimum(g, ngroups - 1)
            pltpu.make_async_copy(x_hbm.at[ridx_of(gcl)], buf, sem).start()

        def wait(buf, sem):
            pltpu.make_async_copy(x_hbm.at[ridx_of(0)], buf, sem).wait()

        def merge(ma, ba, mb, bb):
            m = jnp.maximum(ma, mb)
            bsel = jnp.where(mb > ma, bb, ba)
            btie = jnp.minimum(ba, bb)
            return m, jnp.where(ma == mb, btie, bsel)

        def compute(g, rows_v, carry):
            cc, vc = carry
            t = t0 + g * 16
            tvec = t + io
            preds = jnp.zeros((16,), jnp.int32)
            for gr in range(16):
                def chunk_body(jj, carry2):
                    m0, b0, m1, b1, m2, b2, m3, b3 = carry2
                    base = jj * 4
                    v0 = rows_v[gr, pl.ds((base + 0) * 16, 16)]
                    v1 = rows_v[gr, pl.ds((base + 1) * 16, 16)]
                    v2 = rows_v[gr, pl.ds((base + 2) * 16, 16)]
                    v3 = rows_v[gr, pl.ds((base + 3) * 16, 16)]
                    b0 = jnp.where(v0 > m0, base + 0, b0)
                    m0 = jnp.maximum(m0, v0)
                    b1 = jnp.where(v1 > m1, base + 1, b1)
                    m1 = jnp.maximum(m1, v1)
                    b2 = jnp.where(v2 > m2, base + 2, b2)
                    m2 = jnp.maximum(m2, v2)
                    b3 = jnp.where(v3 > m3, base + 3, b3)
                    m3 = jnp.maximum(m3, v3)
                    return m0, b0, m1, b1, m2, b2, m3, b3
                ninf = jnp.full((16,), -jnp.inf, jnp.float32)
                zi = jnp.zeros((16,), jnp.int32)
                m0, b0, m1, b1, m2, b2, m3, b3 = lax.fori_loop(
                    0, nchunk // 4, chunk_body,
                    (ninf, zi, ninf, zi, ninf, zi, ninf, zi),
                )
                m0, b0 = merge(m0, b0, m1, b1)
                m2, b2 = merge(m2, b2, m3, b3)
                m, bi = merge(m0, b0, m2, b2)
                rm = _all_lanes(m, jnp.maximum)
                cand = jnp.where(
                    m == rm, (bi * 16 + io).astype(jnp.float32), float(V)
                )
                p = _all_lanes(cand, jnp.minimum).astype(jnp.int32)
                preds = jnp.where(io == gr, p, preds)
            tg = tgtbuf[pl.ds(jnp.minimum(t, T), 16)]
            t1_eff = jnp.where(g < ngroups, t1, jnp.int32(-1))
            valid = tvec < t1_eff
            cc = cc + jnp.where(valid & (preds == tg), 1, 0)
            vc = vc + jnp.where(valid, 1, 0)
            return cc, vc

        z = jnp.zeros((16,), jnp.int32)
        cc, vc = z, z

        @pl.when(ngroups > 0)
        def _work():
            pltpu.sync_copy(tgtp_hbm.at[b], tgtbuf)
            start(0, rows_a, sem_a)
            start(1, rows_b, sem_b)
            npairs = lax.div(ngroups + 1, 2)

            def pbody(i, carry):
                g = 2 * i
                wait(rows_a, sem_a)
                carry = compute(g, rows_a, carry)
                start(g + 2, rows_a, sem_a)
                wait(rows_b, sem_b)
                carry = compute(g + 1, rows_b, carry)
                start(g + 3, rows_b, sem_b)
                return carry

            cc2, vc2 = lax.fori_loop(0, npairs, pbody, (z, z))
            wait(rows_a, sem_a)
            wait(rows_b, sem_b)
            ccbuf[...] = cc2
            vcbuf[...] = vc2

        @pl.when(ngroups == 0)
        def _idle():
            ccbuf[...] = z
            vcbuf[...] = z

        pltpu.sync_copy(ccbuf, cc_out.at[wid])
        pltpu.sync_copy(vcbuf, vc_out.at[wid])

    return sck(x2d, tgt_pad, aux)


_R_TC = 2.8e12
_R_SC = 1.2e12
_SC_FIXED = 5e-6


def kernel(outputs, tokens, tokens_lens):
    T, B, V = outputs.shape
    lens = (tokens_lens + 1).astype(jnp.int32)
    tgt = jnp.concatenate([tokens[1:], tokens[-1:]], axis=0)

    cands = jnp.arange(_TB, T + 1, _TB, dtype=jnp.int32)
    tc_cost = cands.astype(jnp.float32) * (B * V * 4) / _R_TC
    sc_rows = jnp.sum(
        jnp.maximum(lens[None, :] - cands[:, None], 0), axis=1
    ).astype(jnp.float32)
    sc_cost = jnp.where(sc_rows > 0, sc_rows * (V * 4) / _R_SC + _SC_FIXED, 0.0)
    s = cands[jnp.argmin(jnp.maximum(tc_cost, sc_cost))]

    aux_tc = jnp.concatenate(
        [lens, s[None], jnp.zeros((7,), jnp.int32)]
    )
    aux_sc = jnp.concatenate(
        [lens, jnp.minimum(s, lens), jnp.zeros((16,), jnp.int32)]
    )

    x2d = outputs.reshape(T * B, V)
    tgt_pad = jnp.concatenate(
        [tgt.T, jnp.zeros((B, 16), jnp.int32)], axis=1
    )

    cc, vc = _sc_counts(x2d, tgt_pad, aux_sc, T, B, V)
    tc = _tc_counts(outputs, tgt, aux_tc)

    c = tc[0] + jnp.sum(cc).astype(jnp.float32)
    m = tc[1] + jnp.sum(vc).astype(jnp.float32)
    return c / m


# speedup vs baseline: 2.7165x; 1.2234x over previous
import functools

import jax
import jax.numpy as jnp
from jax import lax
from jax.experimental import pallas as pl
from jax.experimental.pallas import tpu as pltpu
from jax.experimental.pallas import tpu_sc as plsc

_TB = 256
_NW = 32
_S = 1280


def _tc_kernel(lens_ref, x_ref, tgt_ref, out_ref, c_ref, m_ref):
    j = pl.program_id(0)
    nj = pl.num_programs(0)
    x = x_ref[...]
    TB, B, V = x.shape
    rowmax = jnp.max(x, axis=-1, keepdims=True)
    idx = lax.broadcasted_iota(jnp.int32, x.shape, 2)
    pred = jnp.min(jnp.where(x == rowmax, idx, V), axis=-1)
    tgt = tgt_ref[0]
    t_idx = lax.broadcasted_iota(jnp.int32, (TB, B), 0) + j * TB
    b_idx = lax.broadcasted_iota(jnp.int32, (TB, B), 1)
    lens_v = jnp.zeros((TB, B), jnp.int32)
    for b in range(B):
        lens_v = jnp.where(b_idx == b, lens_ref[b] + 1, lens_v)
    mask = t_idx < jnp.minimum(lens_v, _S)
    c_part = jnp.sum(jnp.where(mask & (pred == tgt), 1.0, 0.0))
    m_part = jnp.sum(jnp.where(mask, 1.0, 0.0))

    @pl.when(j == 0)
    def _init():
        c_ref[0] = 0.0
        m_ref[0] = 0.0

    c_ref[0] += c_part
    m_ref[0] += m_part

    @pl.when(j == nj - 1)
    def _fin():
        out_ref[0] = c_ref[0]
        out_ref[1] = m_ref[0]


def _tc_counts(outputs, tgt, tokens_lens):
    T, B, V = outputs.shape
    nj = _S // _TB
    tgt3 = tgt.reshape(T // _TB, _TB, B)
    grid_spec = pltpu.PrefetchScalarGridSpec(
        num_scalar_prefetch=1,
        grid=(nj,),
        in_specs=[
            pl.BlockSpec((_TB, B, V), lambda j, lens: (j, 0, 0)),
            pl.BlockSpec((1, _TB, B), lambda j, lens: (j, 0, 0)),
        ],
        out_specs=pl.BlockSpec(memory_space=pltpu.SMEM),
        scratch_shapes=[
            pltpu.SMEM((1,), jnp.float32),
            pltpu.SMEM((1,), jnp.float32),
        ],
    )
    return pl.pallas_call(
        _tc_kernel,
        grid_spec=grid_spec,
        out_shape=jax.ShapeDtypeStruct((2,), jnp.float32),
        compiler_params=pltpu.CompilerParams(
            dimension_semantics=("arbitrary",),
        ),
    )(tokens_lens, outputs, tgt3)


def _all_lanes(x, op):
    io = lax.broadcasted_iota(jnp.int32, (16,), 0)
    dnums = lax.GatherDimensionNumbers(
        offset_dims=(), collapsed_slice_dims=(0,), start_index_map=(0,)
    )
    for sh in (8, 4, 2, 1):
        idx = ((io + sh) & 15)[:, None]
        rot = lax.gather(
            x, idx, dnums, (1,),
            mode=lax.GatherScatterMode.PROMISE_IN_BOUNDS,
        )
        x = op(x, rot)
    return x


def _sc_counts(x2d, tgt_flat, lens_pad, T, B, V):
    nchunk = V // 16
    T16 = T + 16

    mesh = plsc.VectorSubcoreMesh(core_axis_name="c", subcore_axis_name="s")

    @functools.partial(
        pl.kernel,
        mesh=mesh,
        out_type=jax.ShapeDtypeStruct((_NW, 2, 16), jnp.int32),
        scratch_types=[
            pltpu.VMEM((16, V), jnp.float32),
            pltpu.VMEM((16, V), jnp.float32),
            pltpu.VMEM((B * (T + 16),), jnp.int32),
            pltpu.VMEM((32,), jnp.int32),
            pltpu.VMEM((2, 16), jnp.int32),
            pltpu.SemaphoreType.DMA,
            pltpu.SemaphoreType.DMA,
        ],
    )
    def sck(x_hbm, tgtf_hbm, lens_hbm, out_hbm,
            rows_a, rows_b, tgtbuf, lensbuf, cntbuf, sem_a, sem_b):
        wid = lax.axis_index("s") * 2 + lax.axis_index("c")
        io = lax.broadcasted_iota(jnp.int32, (16,), 0)

        pltpu.sync_copy(lens_hbm, lensbuf)
        len_c, s_c, off = [], [], [jnp.int32(0)]
        for c in range(B):
            lc = lensbuf[pl.ds(c, 16)][0] + 1
            sc = jnp.minimum(jnp.int32(_S), lc)
            tail_pad = lax.div(lc - sc + 15, 16) * 16
            len_c.append(lc)
            s_c.append(sc)
            off.append(off[-1] + tail_pad)
        gtot = lax.div(off[B], 16)
        per_w = lax.div(gtot + (_NW - 1), _NW)
        g0 = wid * per_w
        g1 = jnp.minimum(g0 + per_w, gtot)
        ngw = jnp.maximum(g1 - g0, 0)

        def group_info(g_loc):
            g16 = (g0 + g_loc) * 16
            c = jnp.int32(0)
            for k in range(1, B):
                c = c + jnp.where(g16 >= off[k], 1, 0)
            lsel = jnp.int32(0)
            ssel = jnp.int32(0)
            osel = jnp.int32(0)
            for k in range(B):
                lsel = jnp.where(c == k, len_c[k], lsel)
                ssel = jnp.where(c == k, s_c[k], ssel)
                osel = jnp.where(c == k, off[k], osel)
            tbase = ssel + (g16 - osel)
            return c, tbase, lsel

        def start(g_loc, buf, sem):
            c, tbase, lsel = group_info(jnp.minimum(g_loc, ngw - 1))
            ridx = jnp.minimum(tbase + io, lsel - 1) * B + c
            pltpu.make_async_copy(x_hbm.at[ridx], buf, sem).start()

        def wait(buf, sem):
            c, tbase, lsel = group_info(jnp.int32(0))
            ridx = jnp.minimum(tbase + io, lsel - 1) * B + c
            pltpu.make_async_copy(x_hbm.at[ridx], buf, sem).wait()

        def merge(ma, ba, mb, bb):
            m = jnp.maximum(ma, mb)
            bsel = jnp.where(mb > ma, bb, ba)
            btie = jnp.minimum(ba, bb)
            return m, jnp.where(ma == mb, btie, bsel)

        def compute(g_loc, rows_v, carry):
            cc, vc = carry
            c, tbase, lsel = group_info(jnp.minimum(g_loc, ngw - 1))
            tvec = tbase + io
            preds = jnp.zeros((16,), jnp.int32)
            for gr in range(16):
                def chunk_body(jj, carry2):
                    m0, b0, m1, b1, m2, b2, m3, b3 = carry2
                    base = jj * 4
                    v0 = rows_v[gr, pl.ds((base + 0) * 16, 16)]
                    v1 = rows_v[gr, pl.ds((base + 1) * 16, 16)]
                    v2 = rows_v[gr, pl.ds((base + 2) * 16, 16)]
                    v3 = rows_v[gr, pl.ds((base + 3) * 16, 16)]
                    b0 = jnp.where(v0 > m0, base + 0, b0)
                    m0 = jnp.maximum(m0, v0)
                    b1 = jnp.where(v1 > m1, base + 1, b1)
                    m1 = jnp.maximum(m1, v1)
                    b2 = jnp.where(v2 > m2, base + 2, b2)
                    m2 = jnp.maximum(m2, v2)
                    b3 = jnp.where(v3 > m3, base + 3, b3)
                    m3 = jnp.maximum(m3, v3)
                    return m0, b0, m1, b1, m2, b2, m3, b3
                ninf = jnp.full((16,), -jnp.inf, jnp.float32)
                zi = jnp.zeros((16,), jnp.int32)
                m0, b0, m1, b1, m2, b2, m3, b3 = lax.fori_loop(
                    0, nchunk // 4, chunk_body,
                    (ninf, zi, ninf, zi, ninf, zi, ninf, zi),
                )
                m0, b0 = merge(m0, b0, m1, b1)
                m2, b2 = merge(m2, b2, m3, b3)
                m, bi = merge(m0, b0, m2, b2)
                rm = _all_lanes(m, jnp.maximum)
                cand = jnp.where(
                    m == rm, (bi * 16 + io).astype(jnp.float32), float(V)
                )
                p = _all_lanes(cand, jnp.minimum).astype(jnp.int32)
                preds = jnp.where(io == gr, p, preds)
            tg = tgtbuf[pl.ds(c * T16 + tbase, 16)]
            l_eff = jnp.where(g_loc < ngw, lsel, jnp.int32(-1))
            valid = tvec < l_eff
            cc = cc + jnp.where(valid & (preds == tg), 1, 0)
            vc = vc + jnp.where(valid, 1, 0)
            return cc, vc

        z = jnp.zeros((16,), jnp.int32)

        @pl.when(ngw > 0)
        def _work():
            pltpu.sync_copy(tgtf_hbm, tgtbuf)
            start(jnp.int32(0), rows_a, sem_a)
            start(jnp.int32(1), rows_b, sem_b)
            npairs = lax.div(ngw + 1, 2)

            def pbody(i, carry):
                g = 2 * i
                wait(rows_a, sem_a)
                carry = compute(g, rows_a, carry)
                start(g + 2, rows_a, sem_a)
                wait(rows_b, sem_b)
                carry = compute(g + 1, rows_b, carry)
                start(g + 3, rows_b, sem_b)
                return carry

            cc2, vc2 = lax.fori_loop(0, npairs, pbody, (z, z))
            wait(rows_a, sem_a)
            wait(rows_b, sem_b)
            cntbuf[0] = cc2
            cntbuf[1] = vc2

        @pl.when(ngw == 0)
        def _idle():
            cntbuf[0] = z
            cntbuf[1] = z

        pltpu.sync_copy(cntbuf, out_hbm.at[wid])

    return sck(x2d, tgt_flat, lens_pad)


def kernel(outputs, tokens, tokens_lens):
    T, B, V = outputs.shape
    lens_pad = jnp.pad(tokens_lens.astype(jnp.int32), (0, 24))
    tgt = jnp.concatenate([tokens[1:], tokens[-1:]], axis=0)
    tgt_flat = jnp.pad(tgt.T, ((0, 0), (0, 16))).reshape(B * (T + 16))
    x2d = outputs.reshape(T * B, V)

    sc = _sc_counts(x2d, tgt_flat, lens_pad, T, B, V)
    tc = _tc_counts(outputs, tgt, tokens_lens)

    scs = jnp.sum(sc, axis=(0, 2)).astype(jnp.float32)
    return (tc[0] + scs[0]) / (tc[1] + scs[1])


# speedup vs baseline: 3.8920x; 1.4327x over previous
import jax
import jax.numpy as jnp
from jax import lax
from jax.experimental import pallas as pl
from jax.experimental.pallas import tpu as pltpu

_TB = 256


def _acc_kernel(lens_ref, x_ref, tgt_ref, out_ref, c_ref, m_ref):
    i = pl.program_id(0)
    nsteps = pl.num_programs(0)
    x = x_ref[...]
    TB, B, V = x.shape
    rowmax = jnp.max(x, axis=-1, keepdims=True)
    idx = lax.broadcasted_iota(jnp.int32, x.shape, 2)
    pred = jnp.min(jnp.where(x == rowmax, idx, V), axis=-1)
    tgt = tgt_ref[0]
    t_idx = lax.broadcasted_iota(jnp.int32, (TB, B), 0) + i * TB
    b_idx = lax.broadcasted_iota(jnp.int32, (TB, B), 1)
    lens_v = jnp.zeros((TB, B), jnp.int32)
    for b in range(B):
        lens_v = jnp.where(b_idx == b, lens_ref[b] + 1, lens_v)
    mask = t_idx < lens_v
    c_part = jnp.sum(jnp.where(mask & (pred == tgt), 1.0, 0.0))
    m_part = jnp.sum(jnp.where(mask, 1.0, 0.0))

    @pl.when(i == 0)
    def _init():
        c_ref[0] = 0.0
        m_ref[0] = 0.0

    c_ref[0] += c_part
    m_ref[0] += m_part

    @pl.when(i == nsteps - 1)
    def _fin():
        out_ref[0] = c_ref[0] / m_ref[0]


def kernel(outputs, tokens, tokens_lens):
    T, B, V = outputs.shape
    tgt = jnp.concatenate([tokens[1:], tokens[-1:]], axis=0)
    ntb = T // _TB
    tgt3 = tgt.reshape(ntb, _TB, B)
    grid_spec = pltpu.PrefetchScalarGridSpec(
        num_scalar_prefetch=1,
        grid=(ntb,),
        in_specs=[
            pl.BlockSpec((_TB, B, V), lambda i, lens: (i, 0, 0)),
            pl.BlockSpec((1, _TB, B), lambda i, lens: (i, 0, 0)),
        ],
        out_specs=pl.BlockSpec(memory_space=pltpu.SMEM),
        scratch_shapes=[
            pltpu.SMEM((1,), jnp.float32),
            pltpu.SMEM((1,), jnp.float32),
        ],
    )
    acc = pl.pallas_call(
        _acc_kernel,
        grid_spec=grid_spec,
        out_shape=jax.ShapeDtypeStruct((1,), jnp.float32),
        compiler_params=pltpu.CompilerParams(
            dimension_semantics=("arbitrary",),
        ),
    )(tokens_lens, outputs, tgt3)
    return acc[0]
